# baseline (speedup 1.0000x reference)
import functools

import jax
import jax.numpy as jnp
from jax import lax
from jax.experimental import pallas as pl
from jax.experimental.pallas import tpu as pltpu
from jax.experimental.pallas import tpu_sc as plsc

BATCH = 16384
CLASSES = 1000
LANES = 16
NW = 32
SC_ROWS = 5120
TC_ROWS = BATCH - SC_ROWS
BPW = SC_ROWS // NW
GR = 16
CG = LANES
NG = BPW // GR
NCH = CLASSES // LANES
TAIL = CLASSES - NCH * LANES
ROW_BLOCK = 1024


def _sc_loss_partials(pred, target, smooth):
    mesh = plsc.VectorSubcoreMesh(core_axis_name="c", subcore_axis_name="s")

    @functools.partial(
        pl.kernel,
        mesh=mesh,
        compiler_params=pltpu.CompilerParams(
            use_tc_tiling_on_sc=True, needs_layout_passes=False),
        out_type=jax.ShapeDtypeStruct((NW * LANES,), jnp.float32),
        scratch_types=[
            pltpu.VMEM((BPW,), jnp.int32),
            pltpu.VMEM((BPW,), jnp.float32),
            pltpu.VMEM((GR, CLASSES), jnp.float32),
            pltpu.VMEM((GR, CLASSES), jnp.float32),
            pltpu.VMEM((LANES,), jnp.float32),
            pltpu.SemaphoreType.DMA,
            pltpu.SemaphoreType.DMA,
        ],
    )
    def body(pred_hbm, tgt_hbm, sm_hbm, out_hbm,
             tgt_v, sm_v, buf_a, buf_b, acc_v, sem_a, sem_b):
        wid = lax.axis_index("s") * 2 + lax.axis_index("c")
        base = TC_ROWS + wid * BPW
        cbase = wid * BPW
        pltpu.sync_copy(tgt_hbm.at[pl.ds(base, BPW)], tgt_v)
        pltpu.sync_copy(sm_hbm.at[pl.ds(base, BPW)], sm_v)
        lane = lax.iota(jnp.int32, LANES)
        onehots = [(lane == r).astype(jnp.float32) for r in range(CG)]
        last_row0 = base + (NG - 1) * GR

        def compute16(cg, off, buf, carry):
            vecacc, sacc = carry
            t = tgt_v[pl.ds(cg * CG, CG)]
            s = jnp.maximum(sm_v[pl.ds(cg * CG, CG)], 0.1)
            fill = s * (1.0 / (CLASSES - 1))
            p = plsc.load_gather(buf, [off + lane, t])
            vecacc = vecacc + (fill + s - 1.0) * p
            rs_vec = jnp.zeros((LANES,), jnp.float32)
            for r in range(CG):
                rv0 = buf[off + r, pl.ds(0, LANES)]
                rv1 = buf[off + r, pl.ds(LANES, LANES)]
                rv2 = buf[off + r, pl.ds(2 * LANES, LANES)]
                rv3 = buf[off + r, pl.ds(3 * LANES, LANES)]
                for k in range(4, NCH - 3, 4):
                    rv0 = rv0 + buf[off + r, pl.ds(k * LANES, LANES)]
                    rv1 = rv1 + buf[off + r, pl.ds((k + 1) * LANES, LANES)]
                    rv2 = rv2 + buf[off + r, pl.ds((k + 2) * LANES, LANES)]
                    rv3 = rv3 + buf[off + r, pl.ds((k + 3) * LANES, LANES)]
                for k in range(NCH - (NCH - 4) % 4, NCH):
                    rv0 = rv0 + buf[off + r, pl.ds(k * LANES, LANES)]
                tail = plsc.load_gather(
                    buf,
                    [jnp.full((LANES,), off + r, jnp.int32),
                     jnp.minimum(NCH * LANES + lane, CLASSES - 1)])
                rv0 = rv0 + jnp.where(lane < TAIL, tail, 0.0)
                rv = (rv0 + rv1) + (rv2 + rv3)
                rs_vec = rs_vec + jnp.sum(rv) * onehots[r]
            sacc = sacc - jnp.sum(fill * rs_vec)
            return vecacc, sacc

        def compute(g, buf, carry):
            for h in range(GR // CG):
                carry = compute16(g * (GR // CG) + h, h * CG, buf, carry)
            return carry

        pltpu.async_copy(pred_hbm.at[pl.ds(base, GR)], buf_a, sem_a)

        def pair(h, carry):
            g0 = 2 * h
            pltpu.make_async_copy(pred_hbm.at[pl.ds(base, GR)], buf_a,
                                  sem_a).wait()
            pltpu.async_copy(
                pred_hbm.at[pl.ds(base + (g0 + 1) * GR, GR)], buf_b, sem_b)
            carry = compute(g0, buf_a, carry)
            pltpu.make_async_copy(pred_hbm.at[pl.ds(base, GR)], buf_b,
                                  sem_b).wait()
            nxt = jnp.minimum(base + (g0 + 2) * GR, last_row0)
            pltpu.async_copy(pred_hbm.at[pl.ds(nxt, GR)], buf_a, sem_a)
            carry = compute(g0 + 1, buf_b, carry)
            return carry

        vecacc, sacc = lax.fori_loop(
            0, NG // 2, pair,
            (jnp.zeros((LANES,), jnp.float32), jnp.float32(0.0)))
        pltpu.make_async_copy(pred_hbm.at[pl.ds(base, GR)], buf_a,
                              sem_a).wait()
        acc_v[...] = vecacc + sacc * onehots[0]
        pltpu.sync_copy(acc_v, out_hbm.at[pl.ds(wid * LANES, LANES)])

    return body(pred, target, smooth)


def _tc_loss_part(pred, target2d, smooth):
    def body(pred_ref, tgt_ref, sm_ref, out_ref):
        x = pred_ref[...]
        t = tgt_ref[...]
        s = jnp.maximum(sm_ref[...], 0.1)
        fill = s * (1.0 / (CLASSES - 1))
        cols = lax.broadcasted_iota(jnp.int32, (ROW_BLOCK, CLASSES), 1)
        p = jnp.sum(jnp.where(cols == t, x, 0.0), axis=1, keepdims=True)
        rowsum = jnp.sum(x, axis=1, keepdims=True)
        partial = jnp.sum((fill + s - 1.0) * p - fill * rowsum)
        elem0 = jnp.logical_and(
            lax.broadcasted_iota(jnp.int32, (8, 128), 0) == 0,
            lax.broadcasted_iota(jnp.int32, (8, 128), 1) == 0)
        out_ref[...] = jnp.where(elem0, partial, 0.0)

    return pl.pallas_call(
        body,
        grid=(TC_ROWS // ROW_BLOCK,),
        in_specs=[
            pl.BlockSpec((ROW_BLOCK, CLASSES), lambda i: (i, 0)),
            pl.BlockSpec((ROW_BLOCK, 1), lambda i: (i, 0)),
            pl.BlockSpec((ROW_BLOCK, 1), lambda i: (i, 0)),
        ],
        out_specs=pl.BlockSpec((8, 128), lambda i: (i, 0)),
        out_shape=jax.ShapeDtypeStruct((TC_ROWS // ROW_BLOCK * 8, 128),
                                       jnp.float32),
        compiler_params=pltpu.CompilerParams(
            dimension_semantics=("parallel",)),
    )(pred, target2d, smooth)


def _tc_combine(scp, tc_part):
    def body(scp_ref, tcp_ref, out_ref):
        out_ref[0, 0] = jnp.sum(scp_ref[...]) + jnp.sum(tcp_ref[...])

    return pl.pallas_call(
        body,
        in_specs=[
            pl.BlockSpec((NW, LANES), lambda: (0, 0)),
            pl.BlockSpec((TC_ROWS // ROW_BLOCK * 8, 128), lambda: (0, 0)),
        ],
        out_specs=pl.BlockSpec(memory_space=pltpu.SMEM),
        out_shape=jax.ShapeDtypeStruct((1, 1), jnp.float32),
    )(scp, tc_part)


def kernel(pred, target, smoothing):
    sm = smoothing.reshape(BATCH)
    scp = _sc_loss_partials(pred, target, sm)
    tcp = _tc_loss_part(pred, target.reshape(BATCH, 1), smoothing)
    out = _tc_combine(scp.reshape(NW, LANES), tcp)
    return out[0, 0]


# speedup vs baseline: 1.1396x; 1.1396x over previous
import functools

import jax
import jax.numpy as jnp
from jax import lax
from jax.experimental import pallas as pl
from jax.experimental.pallas import tpu as pltpu
from jax.experimental.pallas import tpu_sc as plsc

BATCH = 16384
CLASSES = 1000
LANES = 16
NW = 32
SC_ROWS = 3072
TC_ROWS = BATCH - SC_ROWS
BPW = SC_ROWS // NW
GR = 16
CG = LANES
NG = BPW // GR
NCH = CLASSES // LANES
TAIL = CLASSES - NCH * LANES
ROW_BLOCK = 1024


def _sc_loss_partials(pred, target, smooth):
    mesh = plsc.VectorSubcoreMesh(core_axis_name="c", subcore_axis_name="s")

    @functools.partial(
        pl.kernel,
        mesh=mesh,
        compiler_params=pltpu.CompilerParams(
            use_tc_tiling_on_sc=True, needs_layout_passes=False),
        out_type=jax.ShapeDtypeStruct((NW * LANES,), jnp.float32),
        scratch_types=[
            pltpu.VMEM((BPW,), jnp.int32),
            pltpu.VMEM((BPW,), jnp.float32),
            pltpu.VMEM((GR, CLASSES), jnp.float32),
            pltpu.VMEM((GR, CLASSES), jnp.float32),
            pltpu.VMEM((LANES,), jnp.float32),
            pltpu.SemaphoreType.DMA,
            pltpu.SemaphoreType.DMA,
        ],
    )
    def body(pred_hbm, tgt_hbm, sm_hbm, out_hbm,
             tgt_v, sm_v, buf_a, buf_b, acc_v, sem_a, sem_b):
        wid = lax.axis_index("s") * 2 + lax.axis_index("c")
        base = TC_ROWS + wid * BPW
        cbase = wid * BPW
        pltpu.sync_copy(tgt_hbm.at[pl.ds(base, BPW)], tgt_v)
        pltpu.sync_copy(sm_hbm.at[pl.ds(base, BPW)], sm_v)
        lane = lax.iota(jnp.int32, LANES)
        onehots = [(lane == r).astype(jnp.float32) for r in range(CG)]
        last_row0 = base + (NG - 1) * GR

        def compute16(cg, off, buf, carry):
            vecacc, sacc = carry
            t = tgt_v[pl.ds(cg * CG, CG)]
            s = jnp.maximum(sm_v[pl.ds(cg * CG, CG)], 0.1)
            fill = s * (1.0 / (CLASSES - 1))
            p = plsc.load_gather(buf, [off + lane, t])
            vecacc = vecacc + (fill + s - 1.0) * p
            rs_vec = jnp.zeros((LANES,), jnp.float32)
            for r in range(CG):
                rv0 = buf[off + r, pl.ds(0, LANES)]
                rv1 = buf[off + r, pl.ds(LANES, LANES)]
                rv2 = buf[off + r, pl.ds(2 * LANES, LANES)]
                rv3 = buf[off + r, pl.ds(3 * LANES, LANES)]
                for k in range(4, NCH - 3, 4):
                    rv0 = rv0 + buf[off + r, pl.ds(k * LANES, LANES)]
                    rv1 = rv1 + buf[off + r, pl.ds((k + 1) * LANES, LANES)]
                    rv2 = rv2 + buf[off + r, pl.ds((k + 2) * LANES, LANES)]
                    rv3 = rv3 + buf[off + r, pl.ds((k + 3) * LANES, LANES)]
                for k in range(NCH - (NCH - 4) % 4, NCH):
                    rv0 = rv0 + buf[off + r, pl.ds(k * LANES, LANES)]
                tail = plsc.load_gather(
                    buf,
                    [jnp.full((LANES,), off + r, jnp.int32),
                     jnp.minimum(NCH * LANES + lane, CLASSES - 1)])
                rv0 = rv0 + jnp.where(lane < TAIL, tail, 0.0)
                rv = (rv0 + rv1) + (rv2 + rv3)
                rs_vec = rs_vec + jnp.sum(rv) * onehots[r]
            sacc = sacc - jnp.sum(fill * rs_vec)
            return vecacc, sacc

        def compute(g, buf, carry):
            for h in range(GR // CG):
                carry = compute16(g * (GR // CG) + h, h * CG, buf, carry)
            return carry

        pltpu.async_copy(pred_hbm.at[pl.ds(base, GR)], buf_a, sem_a)

        def pair(h, carry):
            g0 = 2 * h
            pltpu.make_async_copy(pred_hbm.at[pl.ds(base, GR)], buf_a,
                                  sem_a).wait()
            pltpu.async_copy(
                pred_hbm.at[pl.ds(base + (g0 + 1) * GR, GR)], buf_b, sem_b)
            carry = compute(g0, buf_a, carry)
            pltpu.make_async_copy(pred_hbm.at[pl.ds(base, GR)], buf_b,
                                  sem_b).wait()
            nxt = jnp.minimum(base + (g0 + 2) * GR, last_row0)
            pltpu.async_copy(pred_hbm.at[pl.ds(nxt, GR)], buf_a, sem_a)
            carry = compute(g0 + 1, buf_b, carry)
            return carry

        vecacc, sacc = lax.fori_loop(
            0, NG // 2, pair,
            (jnp.zeros((LANES,), jnp.float32), jnp.float32(0.0)))
        pltpu.make_async_copy(pred_hbm.at[pl.ds(base, GR)], buf_a,
                              sem_a).wait()
        acc_v[...] = vecacc + sacc * onehots[0]
        pltpu.sync_copy(acc_v, out_hbm.at[pl.ds(wid * LANES, LANES)])

    return body(pred, target, smooth)


def _tc_loss_part(pred, target2d, smooth):
    def body(pred_ref, tgt_ref, sm_ref, out_ref):
        x = pred_ref[...]
        t = tgt_ref[...]
        s = jnp.maximum(sm_ref[...], 0.1)
        fill = s * (1.0 / (CLASSES - 1))
        cols = lax.broadcasted_iota(jnp.int32, (ROW_BLOCK, CLASSES), 1)
        p = jnp.sum(jnp.where(cols == t, x, 0.0), axis=1, keepdims=True)
        rowsum = jnp.sum(x, axis=1, keepdims=True)
        partial = jnp.sum((fill + s - 1.0) * p - fill * rowsum)
        elem0 = jnp.logical_and(
            lax.broadcasted_iota(jnp.int32, (8, 128), 0) == 0,
            lax.broadcasted_iota(jnp.int32, (8, 128), 1) == 0)
        out_ref[...] = jnp.where(elem0, partial, 0.0)

    return pl.pallas_call(
        body,
        grid=(TC_ROWS // ROW_BLOCK,),
        in_specs=[
            pl.BlockSpec((ROW_BLOCK, CLASSES), lambda i: (i, 0)),
            pl.BlockSpec((ROW_BLOCK, 1), lambda i: (i, 0)),
            pl.BlockSpec((ROW_BLOCK, 1), lambda i: (i, 0)),
        ],
        out_specs=pl.BlockSpec((8, 128), lambda i: (i, 0)),
        out_shape=jax.ShapeDtypeStruct((TC_ROWS // ROW_BLOCK * 8, 128),
                                       jnp.float32),
        compiler_params=pltpu.CompilerParams(
            dimension_semantics=("parallel",)),
    )(pred, target2d, smooth)


def _tc_combine(scp, tc_part):
    def body(scp_ref, tcp_ref, out_ref):
        out_ref[0, 0] = jnp.sum(scp_ref[...]) + jnp.sum(tcp_ref[...])

    return pl.pallas_call(
        body,
        in_specs=[
            pl.BlockSpec((NW, LANES), lambda: (0, 0)),
            pl.BlockSpec((TC_ROWS // ROW_BLOCK * 8, 128), lambda: (0, 0)),
        ],
        out_specs=pl.BlockSpec(memory_space=pltpu.SMEM),
        out_shape=jax.ShapeDtypeStruct((1, 1), jnp.float32),
    )(scp, tc_part)


def kernel(pred, target, smoothing):
    sm = smoothing.reshape(BATCH)
    scp = _sc_loss_partials(pred, target, sm)
    tcp = _tc_loss_part(pred, target.reshape(BATCH, 1), smoothing)
    out = _tc_combine(scp.reshape(NW, LANES), tcp)
    return out[0, 0]
